# baseline (speedup 1.0000x reference)
import functools

import jax
import jax.numpy as jnp
from jax import lax
from jax.experimental import pallas as pl
from jax.experimental.pallas import tpu as pltpu
from jax.experimental.pallas import tpu_sc as plsc

f32 = jnp.float32
bf16 = jnp.bfloat16

N = 100000
NP = 100096
E = 1600000
CH = 12800
EP = CH * 128
NW = 32
CPT = CH // NW
CPS = CH // 16
GK = 2
SK = 5
ROWS_G = GK * 128
ROWS_S = SK * 128
NG_G = CPT // GK
NG_C0 = 105
NG_C1 = 295
NG_FS = CPS // SK
NG_ES = CPT // SK
SLAB = NP // 16
R = 5
RS = 2

_mesh = plsc.VectorSubcoreMesh(core_axis_name="c", subcore_axis_name="s")
_sc_params = pltpu.CompilerParams(use_tc_tiling_on_sc=False)


def _make_gather(width):
    def body(h_hbm, dsg, hd, hs, *sc):
        idx = sc[0:R]
        bufd = sc[R:2 * R]
        bufs = sc[2 * R:3 * R]
        gsem = sc[3 * R:4 * R]
        osem = sc[4 * R:]
        cid = lax.axis_index("c")
        sid = lax.axis_index("s")

        def run(base, ng):
            def fire(g, b):
                c0 = base + g * GK
                pltpu.sync_copy(dsg.at[pl.ds(c0 * 2, 2 * GK)], idx[b])
                for j in range(GK):
                    sl = pl.ds(j * 128, 128)
                    pltpu.async_copy(h_hbm.at[idx[b].at[2 * j]], bufd[b].at[sl], gsem[b])
                    pltpu.async_copy(h_hbm.at[idx[b].at[2 * j + 1]], bufs[b].at[sl], gsem[b])

            def drain_g(b):
                for j in range(GK):
                    sl = pl.ds(j * 128, 128)
                    pltpu.make_async_copy(h_hbm.at[idx[b].at[2 * j]], bufd[b].at[sl], gsem[b]).wait()
                    pltpu.make_async_copy(h_hbm.at[idx[b].at[2 * j + 1]], bufs[b].at[sl], gsem[b]).wait()

            def out_descrs(g, b):
                c0 = base + g * GK
                sl = pl.ds(c0 * 128, ROWS_G)
                return (pltpu.make_async_copy(bufd[b], hd.at[sl], osem[b]),
                        pltpu.make_async_copy(bufs[b], hs.at[sl], osem[b]))

            for g0 in range(3):
                fire(g0, g0)

            def outer(a, carry):
                for b in range(R):
                    g = a * R + b
                    b2 = (b + 3) % R

                    @pl.when(g >= 2)
                    def _():
                        for d in out_descrs(g - 2, b2):
                            d.wait()

                    @pl.when(g + 3 < ng)
                    def _():
                        fire(g + 3, b2)

                    drain_g(b)
                    for d in out_descrs(g, b):
                        d.start()
                return carry

            lax.fori_loop(0, ng // R, outer, 0)
            for g in (ng - 2, ng - 1):
                for d in out_descrs(g, g % R):
                    d.wait()

        @pl.when(cid == 0)
        def _():
            run(sid * (NG_C0 * GK), NG_C0)

        @pl.when(cid == 1)
        def _():
            run(16 * NG_C0 * GK + sid * (NG_C1 * GK), NG_C1)

    return functools.partial(
        pl.kernel,
        out_type=[jax.ShapeDtypeStruct((EP, width), f32),
                  jax.ShapeDtypeStruct((EP, width), f32)],
        mesh=_mesh,
        scratch_types=(
            [pltpu.VMEM((2 * GK, 128), jnp.int32) for _ in range(R)]
            + [pltpu.VMEM((ROWS_G, width), f32) for _ in range(2 * R)]
            + [pltpu.SemaphoreType.DMA for _ in range(2 * R)]
        ),
        compiler_params=_sc_params,
    )(body)


_gather16 = _make_gather(16)
_gather32 = _make_gather(32)


def _zero_acc(acc, rows0, sid):
    def zrow(i, c):
        rows0[i, pl.ds(0, 16)] = jnp.zeros((16,), f32)
        return c

    lax.fori_loop(0, ROWS_S, zrow, 0)
    for t in range(9):
        pltpu.sync_copy(rows0, acc.at[pl.ds(sid * SLAB + t * ROWS_S, ROWS_S)])
    rem = SLAB - 9 * ROWS_S
    pltpu.sync_copy(rows0.at[pl.ds(0, rem)],
                    acc.at[pl.ds(sid * SLAB + 9 * ROWS_S, rem)])


def _scatter_ring(h3_at, dsts, acc, idx, rows, lsem, ssem, base, ng):
    def load(g, b):
        c0 = base + g * SK
        pltpu.sync_copy(dsts.at[pl.ds(c0, SK)], idx[b])
        pltpu.async_copy(h3_at(c0), rows[b], lsem[b])

    def rows_descr(g, b):
        c0 = base + g * SK
        return pltpu.make_async_copy(h3_at(c0), rows[b], lsem[b])

    def fire_scatter(b):
        for j in range(SK):
            sl = pl.ds(j * 128, 128)
            pltpu.async_copy(rows[b].at[sl], acc.at[idx[b].at[j]], ssem[b], add=True)

    def drain_scatter(b):
        for j in range(SK):
            sl = pl.ds(j * 128, 128)
            pltpu.make_async_copy(rows[b].at[sl], acc.at[idx[b].at[j]], ssem[b]).wait()

    load(0, 0)

    def outer(a, carry):
        for b in range(RS):
            g = a * RS + b
            b2 = 1 - b

            @pl.when(g >= 1)
            def _():
                drain_scatter(b2)

            @pl.when(g + 1 < ng)
            def _():
                load(g + 1, b2)

            rows_descr(g, b).wait()
            fire_scatter(b)
        return carry

    lax.fori_loop(0, ng // RS, outer, 0)
    drain_scatter((ng - 1) % RS)


def _scatter_fs_body(h3s, dsts, out, acc, *sc):
    idx = sc[0:RS]
    rows = sc[RS:2 * RS]
    lsem = sc[2 * RS:3 * RS]
    ssem = sc[3 * RS:]
    cid = lax.axis_index("c")
    sid = lax.axis_index("s")
    _zero_acc(acc, rows[0], sid)
    plsc.subcore_barrier()

    def h3_at(c0):
        return h3s.at[cid, pl.ds(c0 * 128, ROWS_S)]

    _scatter_ring(h3_at, dsts, acc, idx, rows, lsem, ssem, sid * CPS, NG_FS)
    plsc.subcore_barrier()
    pltpu.sync_copy(acc.at[pl.ds(sid * SLAB, SLAB)],
                    out.at[pl.ds(sid * SLAB, SLAB), pl.ds(cid * 16, 16)])


_scatter_fs_call = functools.partial(
    pl.kernel,
    out_type=jax.ShapeDtypeStruct((NP, 32), f32),
    mesh=_mesh,
    scratch_types=(
        [pltpu.VMEM_SHARED((NP, 16), f32)]
        + [pltpu.VMEM((SK, 128), jnp.int32) for _ in range(RS)]
        + [pltpu.VMEM((ROWS_S, 16), f32) for _ in range(RS)]
        + [pltpu.SemaphoreType.DMA for _ in range(2 * RS)]
    ),
    compiler_params=_sc_params,
)(_scatter_fs_body)


def _scatter_es_body(h3, dsts, out0, out1, acc, *sc):
    idx = sc[0:RS]
    rows = sc[RS:2 * RS]
    lsem = sc[2 * RS:3 * RS]
    ssem = sc[3 * RS:]
    cid = lax.axis_index("c")
    sid = lax.axis_index("s")
    _zero_acc(acc, rows[0], sid)
    plsc.subcore_barrier()
    wid = sid * 2 + cid

    def h3_at(c0):
        return h3.at[pl.ds(c0 * 128, ROWS_S)]

    _scatter_ring(h3_at, dsts, acc, idx, rows, lsem, ssem, wid * CPT, NG_ES)
    plsc.subcore_barrier()
    slab = pl.ds(sid * SLAB, SLAB)

    @pl.when(cid == 0)
    def _():
        pltpu.sync_copy(acc.at[slab], out0.at[slab])

    @pl.when(cid == 1)
    def _():
        pltpu.sync_copy(acc.at[slab], out1.at[slab])


_scatter_es_call = functools.partial(
    pl.kernel,
    out_type=[jax.ShapeDtypeStruct((NP, 16), f32),
              jax.ShapeDtypeStruct((NP, 16), f32)],
    mesh=_mesh,
    scratch_types=(
        [pltpu.VMEM_SHARED((NP, 16), f32)]
        + [pltpu.VMEM((SK, 128), jnp.int32) for _ in range(RS)]
        + [pltpu.VMEM((ROWS_S, 16), f32) for _ in range(RS)]
        + [pltpu.SemaphoreType.DMA for _ in range(2 * RS)]
    ),
    compiler_params=_sc_params,
)(_scatter_es_body)


def _mlp_split_tc(hd_ref, hs_ref, wd_ref, wb_ref, b0_ref,
                  w1_ref, b1_ref, w2lo_ref, b2lo_ref, w2hi_ref, b2hi_ref, o_ref):
    hd = hd_ref[...].astype(bf16)
    hs = hs_ref[...].astype(bf16)
    t = (jnp.dot(hd, wd_ref[...], preferred_element_type=f32)
         + jnp.dot(hs, wb_ref[...], preferred_element_type=f32) + b0_ref[...])
    t = jnp.maximum(t, 0.0).astype(bf16)
    t = jnp.maximum(jnp.dot(t, w1_ref[...], preferred_element_type=f32) + b1_ref[...], 0.0)
    t = t.astype(bf16)
    o_ref[0] = jnp.dot(t, w2lo_ref[...], preferred_element_type=f32) + b2lo_ref[...]
    o_ref[1] = jnp.dot(t, w2hi_ref[...], preferred_element_type=f32) + b2hi_ref[...]


def _mlp_last_tc(hd_ref, hs_ref, wd_ref, wb_ref, b0_ref,
                 w1_ref, b1_ref, w2_ref, b2_ref, o_ref):
    hd = hd_ref[...].astype(bf16)
    hs = hs_ref[...].astype(bf16)
    t = (jnp.dot(hd, wd_ref[...], preferred_element_type=f32)
         + jnp.dot(hs, wb_ref[...], preferred_element_type=f32) + b0_ref[...])
    t = jnp.maximum(t, 0.0).astype(bf16)
    t = jnp.maximum(jnp.dot(t, w1_ref[...], preferred_element_type=f32) + b1_ref[...], 0.0)
    t = t.astype(bf16)
    o_ref[...] = jnp.dot(t, w2_ref[...], preferred_element_type=f32) + b2_ref[...]


def _full(shape):
    return pl.BlockSpec(shape, lambda i: tuple(0 for _ in shape))


def _make_mlp_split(nrows, tcols, ocols):
    blk = 1024
    grid = nrows // blk
    return pl.pallas_call(
        _mlp_split_tc,
        grid=(grid,),
        in_specs=[
            pl.BlockSpec((blk, 128), lambda i: (i, 0)),
            pl.BlockSpec((blk, 128), lambda i: (i, 0)),
            _full((128, tcols)),
            _full((128, tcols)),
            _full((1, tcols)),
            _full((tcols, tcols)),
            _full((1, tcols)),
            _full((tcols, ocols)),
            _full((1, ocols)),
            _full((tcols, ocols)),
            _full((1, ocols)),
        ],
        out_specs=pl.BlockSpec((2, blk, ocols), lambda i: (0, i, 0)),
        out_shape=jax.ShapeDtypeStruct((2, nrows, ocols), f32),
    )


_mlp0 = _make_mlp_split(EP // 8, 256, 128)
_mlp1 = _make_mlp_split(EP // 4, 128, 64)

_mlp2 = pl.pallas_call(
    _mlp_last_tc,
    grid=(EP // 4 // 1024,),
    in_specs=[
        pl.BlockSpec((1024, 128), lambda i: (i, 0)),
        pl.BlockSpec((1024, 128), lambda i: (i, 0)),
        _full((128, 128)),
        _full((128, 128)),
        _full((1, 128)),
        _full((128, 128)),
        _full((1, 128)),
        _full((128, 64)),
        _full((1, 64)),
    ],
    out_specs=pl.BlockSpec((1024, 64), lambda i: (i, 0)),
    out_shape=jax.ShapeDtypeStruct((EP // 4, 64), f32),
)


def _add_tc(a_ref, b_ref, o_ref):
    o_ref[...] = a_ref[...] + b_ref[...]


_add_call = pl.pallas_call(
    _add_tc,
    grid=(4,),
    in_specs=[
        pl.BlockSpec((3128, 128), lambda i: (i, 0)),
        pl.BlockSpec((3128, 128), lambda i: (i, 0)),
    ],
    out_specs=pl.BlockSpec((3128, 128), lambda i: (i, 0)),
    out_shape=jax.ShapeDtypeStruct((NP * 16 // 128, 128), f32),
)


def _perm(copies):
    lo = [32 * e + f for e in range(copies) for f in range(16)]
    hi = [32 * e + 16 + f for e in range(copies) for f in range(16)]
    return jnp.array(lo + hi, dtype=jnp.int32)


def _prep_weights(W0, b0, W1, b1, W2, b2, fin, copies, split):
    eye = jnp.eye(copies, dtype=f32)
    wa = W0[:fin]
    wb = W0[fin:]
    wd_bd = jnp.kron(eye, wa - wb).astype(bf16)
    wb_bd = jnp.kron(eye, wb).astype(bf16)
    b0_t = jnp.tile(b0, copies)[None, :]
    w1 = jnp.kron(eye, W1).astype(bf16)
    b1_t = jnp.tile(b1, copies)[None, :]
    w2 = jnp.kron(eye, W2)
    b2_t = jnp.tile(b2, copies)[None, :]
    if split:
        p = _perm(copies)
        w2 = w2[:, p]
        b2_t = b2_t[:, p]
        half = 16 * copies
        return (wd_bd, wb_bd, b0_t, w1, b1_t,
                w2[:, :half].astype(bf16), b2_t[:, :half],
                w2[:, half:].astype(bf16), b2_t[:, half:])
    return (wd_bd, wb_bd, b0_t, w1, b1_t, w2.astype(bf16), b2_t)


def kernel(x, pos, edge_index, batch,
           W0_0, b0_0, W0_1, b0_1, W0_2, b0_2,
           W1_0, b1_0, W1_1, b1_1, W1_2, b1_2,
           W2_0, b2_0, W2_1, b2_1, W2_2, b2_2):
    src = edge_index[0]
    dst = edge_index[1]
    pad_e = EP - E
    dst_g = jnp.pad(dst, (0, pad_e)).reshape(CH, 1, 128)
    src_g = jnp.pad(src, (0, pad_e)).reshape(CH, 1, 128)
    dsg = jnp.concatenate([dst_g, src_g], axis=1).reshape(2 * CH, 128)
    dst_s = jnp.pad(dst, (0, pad_e), constant_values=N).reshape(CH, 128)
    xp = jnp.pad(x, ((0, NP - N), (0, 0)))

    hd, hs = _gather16(xp, dsg)
    w = _prep_weights(W0_0, b0_0, W0_1, b0_1, W0_2, b0_2, 16, 8, True)
    h3s = _mlp0(hd.reshape(EP // 8, 128), hs.reshape(EP // 8, 128), *w)
    h = _scatter_fs_call(h3s.reshape(2, EP, 16), dst_s)

    hd, hs = _gather32(h, dsg)
    w = _prep_weights(W1_0, b1_0, W1_1, b1_1, W1_2, b1_2, 32, 4, True)
    h3s = _mlp1(hd.reshape(EP // 4, 128), hs.reshape(EP // 4, 128), *w)
    h = _scatter_fs_call(h3s.reshape(2, EP, 16), dst_s)

    hd, hs = _gather32(h, dsg)
    w = _prep_weights(W2_0, b2_0, W2_1, b2_1, W2_2, b2_2, 32, 4, False)
    h3 = _mlp2(hd.reshape(EP // 4, 128), hs.reshape(EP // 4, 128), *w)
    acc0, acc1 = _scatter_es_call(h3.reshape(EP, 16), dst_s)
    out = _add_call(acc0.reshape(NP * 16 // 128, 128),
                    acc1.reshape(NP * 16 // 128, 128))
    return out.reshape(NP, 16)[:N]


# speedup vs baseline: 1.5547x; 1.5547x over previous
import functools

import jax
import jax.numpy as jnp
from jax import lax
from jax.experimental import pallas as pl
from jax.experimental.pallas import tpu as pltpu
from jax.experimental.pallas import tpu_sc as plsc

f32 = jnp.float32
bf16 = jnp.bfloat16

N = 100000
NP = 100096
E = 1600000
EP = 1605632
CH = EP // 128
NW = 32
CPT = CH // NW
CPS = CH // 16
GK = 4
SK = 8
ROWS_G = GK * 128
ROWS_S = SK * 128
SLAB = NP // 16
ZR = 782

_mesh = plsc.VectorSubcoreMesh(core_axis_name="c", subcore_axis_name="s")
_sc_params = pltpu.CompilerParams(use_tc_tiling_on_sc=False)


def _gather_body(a_hbm, b_hbm, dstg, srcg, h1,
                 idxd0, idxs0, idxd1, idxs1,
                 bufa0, bufb0, bufa1, bufb1, sem0, sem1):
    cid = lax.axis_index("c")
    sid = lax.axis_index("s")
    wid = sid * 2 + cid
    base = wid * CPT
    idxd = (idxd0, idxd1)
    idxs = (idxs0, idxs1)
    bufa = (bufa0, bufa1)
    bufb = (bufb0, bufb1)
    sem = (sem0, sem1)

    def fire(h, b):
        c0 = base + h * GK
        pltpu.sync_copy(dstg.at[pl.ds(c0, GK)], idxd[b])
        pltpu.sync_copy(srcg.at[pl.ds(c0, GK)], idxs[b])
        for j in range(GK):
            sl = pl.ds(j * 128, 128)
            pltpu.async_copy(a_hbm.at[idxd[b].at[j]], bufa[b].at[sl], sem[b])
            pltpu.async_copy(b_hbm.at[idxs[b].at[j]], bufb[b].at[sl], sem[b])

    def drain(b):
        for j in range(GK):
            sl = pl.ds(j * 128, 128)
            pltpu.make_async_copy(a_hbm.at[idxd[b].at[j]], bufa[b].at[sl], sem[b]).wait()
            pltpu.make_async_copy(b_hbm.at[idxs[b].at[j]], bufb[b].at[sl], sem[b]).wait()

    fire(0, 0)
    fire(1, 1)
    n_groups = CPT // GK

    def outer(g, carry):
        for b in range(2):
            h = g * 2 + b
            drain(b)

            def add_body(i, c):
                for r in range(4):
                    row = i * 4 + r
                    for half in range(2):
                        sl = pl.ds(half * 16, 16)
                        bufa[b][row, sl] = bufa[b][row, sl] + bufb[b][row, sl]
                return c

            lax.fori_loop(0, ROWS_G // 4, add_body, 0)
            c0 = base + h * GK
            pltpu.sync_copy(bufa[b], h1.at[pl.ds(c0 * 128, ROWS_G)])

            @pl.when(h + 2 < n_groups)
            def _():
                fire(h + 2, b)
        return carry

    lax.fori_loop(0, n_groups // 2, outer, 0)


_gather_call = functools.partial(
    pl.kernel,
    out_type=jax.ShapeDtypeStruct((EP, 32), f32),
    mesh=_mesh,
    scratch_types=[
        pltpu.VMEM((GK, 128), jnp.int32),
        pltpu.VMEM((GK, 128), jnp.int32),
        pltpu.VMEM((GK, 128), jnp.int32),
        pltpu.VMEM((GK, 128), jnp.int32),
        pltpu.VMEM((ROWS_G, 32), f32),
        pltpu.VMEM((ROWS_G, 32), f32),
        pltpu.VMEM((ROWS_G, 32), f32),
        pltpu.VMEM((ROWS_G, 32), f32),
        pltpu.SemaphoreType.DMA,
        pltpu.SemaphoreType.DMA,
    ],
    compiler_params=_sc_params,
)(_gather_body)


def _zero_acc(acc, zbuf, sid):
    def zrow(i, c):
        zbuf[i, pl.ds(0, 16)] = jnp.zeros((16,), f32)
        return c

    lax.fori_loop(0, ZR, zrow, 0)
    for t in range(8):
        pltpu.sync_copy(zbuf, acc.at[pl.ds(sid * SLAB + t * ZR, ZR)])


def _scatter_fs_body(h3, dsts, out, acc, idxv, rows, zbuf, sem):
    cid = lax.axis_index("c")
    sid = lax.axis_index("s")
    _zero_acc(acc, zbuf, sid)
    plsc.subcore_barrier()
    base = sid * CPS

    def grp(g, carry):
        c0 = base + g * SK
        pltpu.sync_copy(dsts.at[pl.ds(c0, SK)], idxv)
        pltpu.sync_copy(h3.at[pl.ds(c0 * 128, ROWS_S), pl.ds(cid * 16, 16)], rows)
        for j in range(SK):
            sl = pl.ds(j * 128, 128)
            pltpu.async_copy(rows.at[sl], acc.at[idxv.at[j]], sem, add=True)
        for j in range(SK):
            sl = pl.ds(j * 128, 128)
            pltpu.make_async_copy(rows.at[sl], acc.at[idxv.at[j]], sem).wait()
        return carry

    lax.fori_loop(0, CPS // SK, grp, 0)
    plsc.subcore_barrier()
    pltpu.sync_copy(acc.at[pl.ds(sid * SLAB, SLAB)],
                    out.at[pl.ds(sid * SLAB, SLAB), pl.ds(cid * 16, 16)])


_scatter_fs_call = functools.partial(
    pl.kernel,
    out_type=jax.ShapeDtypeStruct((NP, 32), f32),
    mesh=_mesh,
    scratch_types=[
        pltpu.VMEM_SHARED((NP, 16), f32),
        pltpu.VMEM((SK, 128), jnp.int32),
        pltpu.VMEM((ROWS_S, 16), f32),
        pltpu.VMEM((ZR, 16), f32),
        pltpu.SemaphoreType.DMA,
    ],
    compiler_params=_sc_params,
)(_scatter_fs_body)


def _scatter_es_body(h3, dsts, out0, out1, acc, idxv, rows, zbuf, sem):
    cid = lax.axis_index("c")
    sid = lax.axis_index("s")
    _zero_acc(acc, zbuf, sid)
    plsc.subcore_barrier()
    wid = sid * 2 + cid
    base = wid * CPT

    def grp(g, carry):
        c0 = base + g * SK
        pltpu.sync_copy(dsts.at[pl.ds(c0, SK)], idxv)
        pltpu.sync_copy(h3.at[pl.ds(c0 * 128, ROWS_S)], rows)
        for j in range(SK):
            sl = pl.ds(j * 128, 128)
            pltpu.async_copy(rows.at[sl], acc.at[idxv.at[j]], sem, add=True)
        for j in range(SK):
            sl = pl.ds(j * 128, 128)
            pltpu.make_async_copy(rows.at[sl], acc.at[idxv.at[j]], sem).wait()
        return carry

    lax.fori_loop(0, CPT // SK, grp, 0)
    plsc.subcore_barrier()
    slab = pl.ds(sid * SLAB, SLAB)

    @pl.when(cid == 0)
    def _():
        pltpu.sync_copy(acc.at[slab], out0.at[slab])

    @pl.when(cid == 1)
    def _():
        pltpu.sync_copy(acc.at[slab], out1.at[slab])


_scatter_es_call = functools.partial(
    pl.kernel,
    out_type=[jax.ShapeDtypeStruct((NP, 16), f32),
              jax.ShapeDtypeStruct((NP, 16), f32)],
    mesh=_mesh,
    scratch_types=[
        pltpu.VMEM_SHARED((NP, 16), f32),
        pltpu.VMEM((SK, 128), jnp.int32),
        pltpu.VMEM((ROWS_S, 16), f32),
        pltpu.VMEM((ZR, 16), f32),
        pltpu.SemaphoreType.DMA,
    ],
    compiler_params=_sc_params,
)(_scatter_es_body)


def _table_tc(x_ref, wd_ref, wb_ref, bd_ref, a_ref, b_ref):
    xv = x_ref[...].astype(bf16)
    a_ref[...] = jnp.dot(xv, wd_ref[...], preferred_element_type=f32) + bd_ref[...]
    b_ref[...] = jnp.dot(xv, wb_ref[...], preferred_element_type=f32)


def _make_table(nrows, in_cols, out_cols, grid):
    blk = nrows // grid
    return pl.pallas_call(
        _table_tc,
        grid=(grid,),
        in_specs=[
            pl.BlockSpec((blk, in_cols), lambda i: (i, 0)),
            pl.BlockSpec((in_cols, out_cols), lambda i: (0, 0)),
            pl.BlockSpec((in_cols, out_cols), lambda i: (0, 0)),
            pl.BlockSpec((1, out_cols), lambda i: (0, 0)),
        ],
        out_specs=[
            pl.BlockSpec((blk, out_cols), lambda i: (i, 0)),
            pl.BlockSpec((blk, out_cols), lambda i: (i, 0)),
        ],
        out_shape=[
            jax.ShapeDtypeStruct((nrows, out_cols), f32),
            jax.ShapeDtypeStruct((nrows, out_cols), f32),
        ],
    )


def _mid_tc(h_ref, w1_ref, b1_ref, w2_ref, b2_ref, o_ref):
    t = jnp.maximum(h_ref[...], 0.0).astype(bf16)
    t = jnp.maximum(jnp.dot(t, w1_ref[...], preferred_element_type=f32) + b1_ref[...], 0.0)
    t = t.astype(bf16)
    o_ref[...] = jnp.dot(t, w2_ref[...], preferred_element_type=f32) + b2_ref[...]


def _make_mid(out_cols):
    nrows = EP // 4
    blk = 1024
    grid = nrows // blk
    return pl.pallas_call(
        _mid_tc,
        grid=(grid,),
        in_specs=[
            pl.BlockSpec((blk, 128), lambda i: (i, 0)),
            pl.BlockSpec((128, 128), lambda i: (0, 0)),
            pl.BlockSpec((1, 128), lambda i: (0, 0)),
            pl.BlockSpec((128, out_cols), lambda i: (0, 0)),
            pl.BlockSpec((1, out_cols), lambda i: (0, 0)),
        ],
        out_specs=pl.BlockSpec((blk, out_cols), lambda i: (i, 0)),
        out_shape=jax.ShapeDtypeStruct((nrows, out_cols), f32),
    )


def _add_tc(a_ref, b_ref, o_ref):
    o_ref[...] = a_ref[...] + b_ref[...]


_add_call = pl.pallas_call(
    _add_tc,
    grid=(4,),
    in_specs=[
        pl.BlockSpec((3128, 128), lambda i: (i, 0)),
        pl.BlockSpec((3128, 128), lambda i: (i, 0)),
    ],
    out_specs=pl.BlockSpec((3128, 128), lambda i: (i, 0)),
    out_shape=jax.ShapeDtypeStruct((NP * 16 // 128, 128), f32),
)

_table0 = _make_table(NP // 8, 128, 256, 4)
_table12 = _make_table(NP // 4, 128, 128, 4)
_mid32 = _make_mid(128)
_mid16 = _make_mid(64)


def _prep_first_layer(W, b, fin, copies):
    wa = W[:fin]
    wb = W[fin:]
    eye = jnp.eye(copies, dtype=f32)
    wd_bd = jnp.kron(eye, wa - wb).astype(bf16)
    wb_bd = jnp.kron(eye, wb).astype(bf16)
    b_t = jnp.tile(b, copies)[None, :]
    return wd_bd, wb_bd, b_t


def _prep_mid(W1, b1, W2, b2):
    eye = jnp.eye(4, dtype=f32)
    return (jnp.kron(eye, W1).astype(bf16), jnp.tile(b1, 4)[None, :],
            jnp.kron(eye, W2).astype(bf16), jnp.tile(b2, 4)[None, :])


def kernel(x, pos, edge_index, batch,
           W0_0, b0_0, W0_1, b0_1, W0_2, b0_2,
           W1_0, b1_0, W1_1, b1_1, W1_2, b1_2,
           W2_0, b2_0, W2_1, b2_1, W2_2, b2_2):
    src = edge_index[0]
    dst = edge_index[1]
    pad_e = EP - E
    dst_g = jnp.pad(dst, (0, pad_e)).reshape(CH, 128)
    src_g = jnp.pad(src, (0, pad_e)).reshape(CH, 128)
    dst_s = jnp.pad(dst, (0, pad_e), constant_values=N).reshape(CH, 128)
    xp = jnp.pad(x, ((0, NP - N), (0, 0)))

    wd, wb, bt = _prep_first_layer(W0_0, b0_0, 16, 8)
    a_t, b_t = _table0(xp.reshape(NP // 8, 128), wd, wb, bt)
    h1 = _gather_call(a_t.reshape(NP, 32), b_t.reshape(NP, 32), dst_g, src_g)
    m1, bm1, m2, bm2 = _prep_mid(W0_1, b0_1, W0_2, b0_2)
    h3 = _mid32(h1.reshape(EP // 4, 128), m1, bm1, m2, bm2)
    h = _scatter_fs_call(h3.reshape(EP, 32), dst_s)

    wd, wb, bt = _prep_first_layer(W1_0, b1_0, 32, 4)
    a_t, b_t = _table12(h.reshape(NP // 4, 128), wd, wb, bt)
    h1 = _gather_call(a_t.reshape(NP, 32), b_t.reshape(NP, 32), dst_g, src_g)
    m1, bm1, m2, bm2 = _prep_mid(W1_1, b1_1, W1_2, b1_2)
    h3 = _mid32(h1.reshape(EP // 4, 128), m1, bm1, m2, bm2)
    h = _scatter_fs_call(h3.reshape(EP, 32), dst_s)

    wd, wb, bt = _prep_first_layer(W2_0, b2_0, 32, 4)
    a_t, b_t = _table12(h.reshape(NP // 4, 128), wd, wb, bt)
    h1 = _gather_call(a_t.reshape(NP, 32), b_t.reshape(NP, 32), dst_g, src_g)
    m1, bm1, m2, bm2 = _prep_mid(W2_1, b2_1, W2_2, b2_2)
    h3 = _mid16(h1.reshape(EP // 4, 128), m1, bm1, m2, bm2)
    acc0, acc1 = _scatter_es_call(h3.reshape(EP, 16), dst_s)
    out = _add_call(acc0.reshape(NP * 16 // 128, 128),
                    acc1.reshape(NP * 16 // 128, 128))
    return out.reshape(NP, 16)[:N]
